# baseline (speedup 1.0000x reference)
import jax
import jax.numpy as jnp
from jax.experimental import pallas as pl
from jax.experimental.pallas import tpu as pltpu

N, C, H, W = 8, 21, 512, 512
CB = 7
NC = C // CB
SMOOTH = 1e-05


def _f1_body(x_ref, t_ref, out_ref, best, bidx, f1sum):
    n = pl.program_id(0)
    c = pl.program_id(1)

    @pl.when(c == 0)
    def _init():
        best[...] = jnp.full((H, W), -jnp.inf, jnp.float32)
        bidx[...] = jnp.zeros((H, W), jnp.int32)

    b = best[...]
    bi = bidx[...]
    for j in range(CB):
        v = x_ref[0, j]
        cls = c * CB + j
        upd = v > b
        b = jnp.where(upd, v, b)
        bi = jnp.where(upd, cls, bi)
    best[...] = b
    bidx[...] = bi

    @pl.when(c == NC - 1)
    def _fin_n():
        tgt = t_ref[0]
        ht = jnp.where(bi == tgt, tgt, C)
        f1s = jnp.float32(0.0)
        for cls in range(C):
            tt = jnp.sum(jnp.where(tgt == cls, 1.0, 0.0))
            pp = jnp.sum(jnp.where(bi == cls, 1.0, 0.0))
            tp = jnp.sum(jnp.where(ht == cls, 1.0, 0.0))
            recall = (tp + SMOOTH) / (tt + SMOOTH)
            precision = (tp + SMOOTH) / (pp + SMOOTH)
            f1s = f1s + 2.0 * recall * precision / (recall + precision)
        prev = jnp.where(n == 0, jnp.float32(0.0), f1sum[0])
        f1sum[0] = prev + f1s

    @pl.when((n == N - 1) & (c == NC - 1))
    def _fin():
        out_ref[0] = jnp.float32(1.0) - f1sum[0] / jnp.float32(N * C)


@jax.jit
def kernel(input, target):
    out = pl.pallas_call(
        _f1_body,
        grid=(N, NC),
        in_specs=[
            pl.BlockSpec((1, CB, H, W), lambda n, c: (n, c, 0, 0)),
            pl.BlockSpec((1, H, W), lambda n, c: (n, 0, 0)),
        ],
        out_specs=pl.BlockSpec(memory_space=pltpu.SMEM),
        out_shape=jax.ShapeDtypeStruct((1,), jnp.float32),
        scratch_shapes=[
            pltpu.VMEM((H, W), jnp.float32),
            pltpu.VMEM((H, W), jnp.int32),
            pltpu.SMEM((1,), jnp.float32),
        ],
        compiler_params=pltpu.CompilerParams(
            dimension_semantics=("arbitrary", "arbitrary"),
        ),
    )(input, target)
    return out[0]


# speedup vs baseline: 1.4267x; 1.4267x over previous
import jax
import jax.numpy as jnp
from jax.experimental import pallas as pl
from jax.experimental.pallas import tpu as pltpu

N, C, H, W = 8, 21, 512, 512
CB = 7
NC = C // CB
SMOOTH = 1e-05


def _f1_body(x_ref, t_ref, out_ref, best, bidx, f1sum):
    n = pl.program_id(0)
    c = pl.program_id(1)

    @pl.when(c == 0)
    def _init():
        best[...] = jnp.full((H, W), -jnp.inf, jnp.float32)
        bidx[...] = jnp.zeros((H, W), jnp.int32)

    b = best[...]
    bi = bidx[...]
    for j in range(CB):
        v = x_ref[0, j]
        cls = c * CB + j
        upd = v > b
        b = jnp.where(upd, v, b)
        bi = jnp.where(upd, cls, bi)
    best[...] = b
    bidx[...] = bi

    @pl.when(c == NC - 1)
    def _fin_n():
        nslab = H // 8
        plane_lists = []
        for s in range(3):
            planes = [jnp.zeros((8, W), jnp.int32) for _ in range(22)]
            acc4 = [jnp.zeros((8, W), jnp.int32) for _ in range(3)]
            pend = 0
            for k in range(nslab):
                r = pl.ds(8 * k, 8)
                if s == 0:
                    v = t_ref[0, r, :]
                else:
                    b_s = bidx[r, :]
                    if s == 1:
                        v = b_s
                    else:
                        t_s = t_ref[0, r, :]
                        v = jnp.where(b_s == t_s, t_s, C)
                g = v >> 3
                nib = jnp.int32(1) << ((v & 7) << 2)
                for gg in range(3):
                    acc4[gg] = acc4[gg] + jnp.where(g == gg, nib, 0)
                pend += 1
                if pend == 15 or k == nslab - 1:
                    for gg in range(3):
                        a = acc4[gg]
                        for c7 in range(8):
                            cc = 8 * gg + c7
                            if cc < 22:
                                planes[cc] = planes[cc] + (
                                    (a >> (4 * c7)) & 15)
                        if k != nslab - 1:
                            acc4[gg] = jnp.zeros((8, W), jnp.int32)
                    pend = 0
            plane_lists.append(planes)

        f1s = jnp.float32(0.0)
        for cls in range(C):
            tt = jnp.sum(plane_lists[0][cls]).astype(jnp.float32)
            pp = jnp.sum(plane_lists[1][cls]).astype(jnp.float32)
            tp = jnp.sum(plane_lists[2][cls]).astype(jnp.float32)
            recall = (tp + SMOOTH) / (tt + SMOOTH)
            precision = (tp + SMOOTH) / (pp + SMOOTH)
            f1s = f1s + 2.0 * recall * precision / (recall + precision)
        prev = jnp.where(n == 0, jnp.float32(0.0), f1sum[0])
        f1sum[0] = prev + f1s

    @pl.when((n == N - 1) & (c == NC - 1))
    def _fin():
        out_ref[0] = jnp.float32(1.0) - f1sum[0] / jnp.float32(N * C)


@jax.jit
def kernel(input, target):
    out = pl.pallas_call(
        _f1_body,
        grid=(N, NC),
        in_specs=[
            pl.BlockSpec((1, CB, H, W), lambda n, c: (n, c, 0, 0)),
            pl.BlockSpec((1, H, W), lambda n, c: (n, 0, 0)),
        ],
        out_specs=pl.BlockSpec(memory_space=pltpu.SMEM),
        out_shape=jax.ShapeDtypeStruct((1,), jnp.float32),
        scratch_shapes=[
            pltpu.VMEM((H, W), jnp.float32),
            pltpu.VMEM((H, W), jnp.int32),
            pltpu.SMEM((1,), jnp.float32),
        ],
        compiler_params=pltpu.CompilerParams(
            dimension_semantics=("arbitrary", "arbitrary"),
        ),
    )(input, target)
    return out[0]


# speedup vs baseline: 1.4429x; 1.0114x over previous
import jax
import jax.numpy as jnp
from jax.experimental import pallas as pl
from jax.experimental.pallas import tpu as pltpu

N, C, H, W = 8, 21, 512, 512
CB = 7
NC = C // CB
STEPS = N * NC
NSLAB = H // 8
RANGES = [(0, 22), (22, 44), (44, NSLAB)]
SMOOTH = 1e-05


def _count_chunk(t_prev, b_prev, planes_ref, k0, k1):
    for s in range(3):
        acc4 = [jnp.zeros((8, W), jnp.int32) for _ in range(3)]
        pend = 0
        for k in range(k0, k1):
            r = pl.ds(8 * k, 8)
            if s == 0:
                v = t_prev[r, :]
            elif s == 1:
                v = b_prev[r, :]
            else:
                b_s = b_prev[r, :]
                t_s = t_prev[r, :]
                v = jnp.where(b_s == t_s, t_s, C)
            g = v >> 3
            nib = jnp.int32(1) << ((v & 7) << 2)
            for gg in range(3):
                acc4[gg] = acc4[gg] + jnp.where(g == gg, nib, 0)
            pend += 1
            if pend == 15 or k == k1 - 1:
                for gg in range(3):
                    a = acc4[gg]
                    for c7 in range(8):
                        cc = 8 * gg + c7
                        if cc < 22:
                            row = pl.ds((s * 22 + cc) * 8, 8)
                            planes_ref[row, :] += (a >> (4 * c7)) & 15
                    if k != k1 - 1:
                        acc4[gg] = jnp.zeros((8, W), jnp.int32)
                pend = 0


def _finish_f1(planes_ref):
    f1s = jnp.float32(0.0)
    for cls in range(C):
        tt = jnp.sum(planes_ref[pl.ds((0 * 22 + cls) * 8, 8), :])
        pp = jnp.sum(planes_ref[pl.ds((1 * 22 + cls) * 8, 8), :])
        tp = jnp.sum(planes_ref[pl.ds((2 * 22 + cls) * 8, 8), :])
        ttf = tt.astype(jnp.float32)
        ppf = pp.astype(jnp.float32)
        tpf = tp.astype(jnp.float32)
        recall = (tpf + SMOOTH) / (ttf + SMOOTH)
        precision = (tpf + SMOOTH) / (ppf + SMOOTH)
        f1s = f1s + 2.0 * recall * precision / (recall + precision)
    planes_ref[...] = jnp.zeros_like(planes_ref)
    return f1s


def _f1_body(x_ref, t_ref, out_ref, best, bidx, t_prev, b_prev,
             planes_ref, f1sum):
    i = pl.program_id(0)
    c = i % NC

    @pl.when(i == 0)
    def _start():
        planes_ref[...] = jnp.zeros_like(planes_ref)
        f1sum[0] = jnp.float32(0.0)

    @pl.when(i < STEPS)
    def _argmax():
        @pl.when(c == 0)
        def _init():
            best[...] = jnp.full((H, W), -jnp.inf, jnp.float32)
            bidx[...] = jnp.zeros((H, W), jnp.int32)

        b = best[...]
        bi = bidx[...]
        for j in range(CB):
            v = x_ref[0, j]
            cls = c * CB + j
            upd = v > b
            b = jnp.where(upd, v, b)
            bi = jnp.where(upd, cls, bi)
        best[...] = b
        bidx[...] = bi

    for phase in range(NC):
        @pl.when((i >= NC) & (i < STEPS) & (c == phase))
        def _deferred(ph=phase):
            k0, k1 = RANGES[ph]
            _count_chunk(t_prev, b_prev, planes_ref, k0, k1)

    @pl.when((i >= NC) & (i < STEPS) & (c == NC - 1))
    def _f1_prev():
        f1sum[0] += _finish_f1(planes_ref)

    @pl.when((i < STEPS) & (c == NC - 1))
    def _snapshot():
        t_prev[...] = t_ref[0]
        b_prev[...] = bidx[...]

    @pl.when(i == STEPS)
    def _epilogue():
        _count_chunk(t_prev, b_prev, planes_ref, 0, NSLAB)
        f1s = _finish_f1(planes_ref)
        tot = f1sum[0] + f1s
        out_ref[0] = jnp.float32(1.0) - tot / jnp.float32(N * C)


@jax.jit
def kernel(input, target):
    def x_map(i):
        ii = jnp.minimum(i, STEPS - 1)
        return (ii // NC, ii % NC, 0, 0)

    def t_map(i):
        return (jnp.minimum(i // NC, N - 1), 0, 0)

    out = pl.pallas_call(
        _f1_body,
        grid=(STEPS + 1,),
        in_specs=[
            pl.BlockSpec((1, CB, H, W), x_map),
            pl.BlockSpec((1, H, W), t_map),
        ],
        out_specs=pl.BlockSpec(memory_space=pltpu.SMEM),
        out_shape=jax.ShapeDtypeStruct((1,), jnp.float32),
        scratch_shapes=[
            pltpu.VMEM((H, W), jnp.float32),
            pltpu.VMEM((H, W), jnp.int32),
            pltpu.VMEM((H, W), jnp.int32),
            pltpu.VMEM((H, W), jnp.int32),
            pltpu.VMEM((3 * 22 * 8, W), jnp.int32),
            pltpu.SMEM((1,), jnp.float32),
        ],
        compiler_params=pltpu.CompilerParams(
            dimension_semantics=("arbitrary",),
        ),
    )(input, target)
    return out[0]


# speedup vs baseline: 1.8801x; 1.3030x over previous
import jax
import jax.numpy as jnp
from jax.experimental import pallas as pl
from jax.experimental.pallas import tpu as pltpu

N, C, H, W = 8, 21, 512, 512
NSLAB = H // 8
SMOOTH = 1e-05


def _count_image(t_prev, b_prev, planes_ref):
    for s in range(3):
        acc4 = [jnp.zeros((8, W), jnp.int32) for _ in range(3)]
        pend = 0
        for k in range(NSLAB):
            r = pl.ds(8 * k, 8)
            if s == 0:
                v = t_prev[r, :]
            elif s == 1:
                v = b_prev[r, :]
            else:
                b_s = b_prev[r, :]
                t_s = t_prev[r, :]
                v = jnp.where(b_s == t_s, t_s, C)
            g = v >> 3
            nib = jnp.int32(1) << ((v & 7) << 2)
            for gg in range(3):
                acc4[gg] = acc4[gg] + jnp.where(g == gg, nib, 0)
            pend += 1
            if pend == 15 or k == NSLAB - 1:
                for gg in range(3):
                    a = acc4[gg]
                    for c7 in range(8):
                        cc = 8 * gg + c7
                        if cc < 22:
                            row = pl.ds((s * 22 + cc) * 8, 8)
                            planes_ref[row, :] += (a >> (4 * c7)) & 15
                    if k != NSLAB - 1:
                        acc4[gg] = jnp.zeros((8, W), jnp.int32)
                pend = 0


def _finish_f1(planes_ref):
    f1s = jnp.float32(0.0)
    for cls in range(C):
        tt = jnp.sum(planes_ref[pl.ds((0 * 22 + cls) * 8, 8), :])
        pp = jnp.sum(planes_ref[pl.ds((1 * 22 + cls) * 8, 8), :])
        tp = jnp.sum(planes_ref[pl.ds((2 * 22 + cls) * 8, 8), :])
        ttf = tt.astype(jnp.float32)
        ppf = pp.astype(jnp.float32)
        tpf = tp.astype(jnp.float32)
        recall = (tpf + SMOOTH) / (ttf + SMOOTH)
        precision = (tpf + SMOOTH) / (ppf + SMOOTH)
        f1s = f1s + 2.0 * recall * precision / (recall + precision)
    planes_ref[...] = jnp.zeros_like(planes_ref)
    return f1s


def _f1_body(x_ref, t_ref, out_ref, t_prev, b_prev, planes_ref, f1sum):
    i = pl.program_id(0)

    @pl.when(i == 0)
    def _start():
        planes_ref[...] = jnp.zeros_like(planes_ref)
        f1sum[0] = jnp.float32(0.0)

    @pl.when(i >= 1)
    def _deferred():
        _count_image(t_prev, b_prev, planes_ref)
        f1s = _finish_f1(planes_ref)
        f1sum[0] += f1s

        @pl.when(i == N)
        def _out():
            out_ref[0] = jnp.float32(1.0) - f1sum[0] / jnp.float32(N * C)

    @pl.when(i < N)
    def _argmax():
        b = x_ref[0, 0]
        bi = jnp.zeros((H, W), jnp.int32)
        for cls in range(1, C):
            v = x_ref[0, cls]
            upd = v > b
            b = jnp.where(upd, v, b)
            bi = jnp.where(upd, cls, bi)
        b_prev[...] = bi
        t_prev[...] = t_ref[0]


@jax.jit
def kernel(input, target):
    def x_map(i):
        return (jnp.minimum(i, N - 1), 0, 0, 0)

    def t_map(i):
        return (jnp.minimum(i, N - 1), 0, 0)

    out = pl.pallas_call(
        _f1_body,
        grid=(N + 1,),
        in_specs=[
            pl.BlockSpec((1, C, H, W), x_map),
            pl.BlockSpec((1, H, W), t_map),
        ],
        out_specs=pl.BlockSpec(memory_space=pltpu.SMEM),
        out_shape=jax.ShapeDtypeStruct((1,), jnp.float32),
        scratch_shapes=[
            pltpu.VMEM((H, W), jnp.int32),
            pltpu.VMEM((H, W), jnp.int32),
            pltpu.VMEM((3 * 22 * 8, W), jnp.int32),
            pltpu.SMEM((1,), jnp.float32),
        ],
        compiler_params=pltpu.CompilerParams(
            dimension_semantics=("arbitrary",),
        ),
    )(input, target)
    return out[0]
